# baseline (speedup 1.0000x reference)
import functools

import jax
import jax.numpy as jnp
from jax import lax
from jax.experimental import pallas as pl
from jax.experimental.pallas import tpu as pltpu
from jax.experimental.pallas import tpu_sc as plsc

_B, _S, _W = 4, 2048, 4096
_R = _B * _S
_NW = 32
_RPW = _R // _NW
_CR = 8
_CC = 1024
_L = 16
_NBUF = 4


@functools.partial(
    pl.kernel,
    out_type=jax.ShapeDtypeStruct((_R, 2 * _W), jnp.float32),
    mesh=plsc.VectorSubcoreMesh(core_axis_name="c", subcore_axis_name="s"),
    scratch_types=(
        [pltpu.VMEM((_CR, _CC), jnp.float32) for _ in range(_NBUF)]
        + [pltpu.VMEM((_CR, 2 * _CC), jnp.float32) for _ in range(_NBUF)]
        + [pltpu.SemaphoreType.DMA for _ in range(2 * _NBUF)]
    ),
    compiler_params=pltpu.CompilerParams(
        needs_layout_passes=False, use_tc_tiling_on_sc=True),
)
def _interp_sc(x_hbm, y_hbm, *refs):
    in_bufs = refs[:_NBUF]
    out_bufs = refs[_NBUF:2 * _NBUF]
    g_sems = refs[2 * _NBUF:3 * _NBUF]
    s_sems = refs[3 * _NBUF:4 * _NBUF]

    wid = lax.axis_index("s") * 2 + lax.axis_index("c")
    row0 = wid * _RPW
    _CPR = _W // _CC
    _NCHUNK = (_RPW // _CR) * _CPR

    evens = lax.iota(jnp.int32, _L) * 2
    rowsel = [jnp.full((_L,), s, jnp.int32) for s in range(_CR)]

    def src_slice(c):
        r = row0 + (c // _CPR) * _CR
        col = (c % _CPR) * _CC
        return x_hbm.at[pl.ds(r, _CR), pl.ds(col, _CC)]

    def dst_slice(c):
        r = row0 + (c // _CPR) * _CR
        col = (c % _CPR) * _CC
        return y_hbm.at[pl.ds(r, _CR), pl.ds(2 * col, 2 * _CC)]

    for b in range(_NBUF):
        pltpu.async_copy(src_slice(b), in_bufs[b], g_sems[b])

    zeros = jnp.zeros((_L,), jnp.float32)

    def zbody(i, carry):
        for b in range(_NBUF):
            for s in range(_CR):
                out_bufs[b][s, pl.ds(i * _L, _L)] = zeros
        return carry

    lax.fori_loop(0, (2 * _CC) // _L, zbody, 0)

    def make_ibody(in_v, out_v):
        def ibody(i, carry):
            for s in range(_CR):
                xv = in_v[s, pl.ds(i * _L, _L)]
                plsc.store_scatter(
                    out_v, [rowsel[s], evens + i * (2 * _L)], xv)
            return carry
        return ibody

    def cbody(cg, carry):
        for b in range(_NBUF):
            in_v, out_v = in_bufs[b], out_bufs[b]
            sg, ss = g_sems[b], s_sems[b]
            cc = cg * _NBUF + b
            pltpu.make_async_copy(src_slice(cc), in_v, sg).wait()
            @pl.when(cc >= _NBUF)
            def _():
                pltpu.make_async_copy(out_v, dst_slice(cc), ss).wait()
            lax.fori_loop(0, _CC // _L, make_ibody(in_v, out_v), 0)
            pltpu.async_copy(out_v, dst_slice(cc), ss)
            @pl.when(cc + _NBUF < _NCHUNK)
            def _():
                pltpu.async_copy(src_slice(cc + _NBUF), in_v, sg)
        return carry

    lax.fori_loop(0, _NCHUNK // _NBUF, cbody, 0)

    for b in range(_NBUF):
        pltpu.make_async_copy(
            out_bufs[b], dst_slice(_NCHUNK - _NBUF + b), s_sems[b]).wait()


def kernel(x):
    y = _interp_sc(x.reshape(_R, _W))
    return y.reshape(_B, _S, 2 * _W)
